# baseline (speedup 1.0000x reference)
import functools

import jax
import jax.numpy as jnp
from jax import lax
from jax.experimental import pallas as pl
from jax.experimental.pallas import tpu as pltpu
from jax.experimental.pallas import tpu_sc as plsc

FEATS = 512
ROWS = 16
NTIMES = 16384
NC, NS, L = 2, 16, 16
NW = NC * NS
CPW = NTIMES // NW
NGROUPS = CPW // L

_mesh = plsc.VectorSubcoreMesh(core_axis_name="c", subcore_axis_name="s")


@functools.partial(
    pl.kernel,
    mesh=_mesh,
    compiler_params=pltpu.CompilerParams(needs_layout_passes=False),
    out_type=jax.ShapeDtypeStruct((NTIMES * FEATS,), jnp.float32),
    scratch_types=[
        pltpu.VMEM((CPW,), jnp.float32),
        pltpu.VMEM((ROWS * FEATS,), jnp.float32),
        pltpu.VMEM((L * FEATS,), jnp.float32),
    ],
)
def _sc_interp(times_hbm, table_hbm, out_hbm, times_v, table_v, out_v):
    wid = lax.axis_index("s") * NC + lax.axis_index("c")
    base = wid * CPW
    pltpu.sync_copy(times_hbm.at[pl.ds(base, CPW)], times_v)
    pltpu.sync_copy(table_hbm, table_v)

    orow = lax.iota(jnp.int32, L) * FEATS

    def group_body(g, carry):
        t = times_v[pl.ds(g * L, L)]
        data = t * float(ROWS)
        li = jnp.clip(data, 0.0, float(ROWS - 1)).astype(jnp.int32)
        ri = jnp.minimum(li + 1, ROWS - 1)
        lw = data - li.astype(jnp.float32)
        rw = 1.0 - lw
        lbase = li * FEATS
        rbase = ri * FEATS

        def col_body(c, carry2):
            cs = jnp.full((L,), c, dtype=jnp.int32)
            le = plsc.load_gather(table_v, [lbase + cs])
            re = plsc.load_gather(table_v, [rbase + cs])
            o = rw * le + lw * re
            plsc.store_scatter(out_v, [orow + cs], o)
            return carry2

        lax.fori_loop(0, FEATS, col_body, 0, unroll=4)
        pltpu.sync_copy(out_v, out_hbm.at[pl.ds((base + g * L) * FEATS, L * FEATS)])
        return carry

    lax.fori_loop(0, NGROUPS, group_body, 0)


def kernel(times, table):
    out = _sc_interp(times, table.reshape(ROWS * FEATS))
    return out.reshape(NTIMES, FEATS)


# speedup vs baseline: 1.7964x; 1.7964x over previous
import functools

import jax
import jax.numpy as jnp
from jax import lax
from jax.experimental import pallas as pl
from jax.experimental.pallas import tpu as pltpu
from jax.experimental.pallas import tpu_sc as plsc

FEATS = 512
ROWS = 16
NTIMES = 16384
NC, NS, L = 2, 16, 16
NW = NC * NS
CPW = NTIMES // NW
NGROUPS = CPW // L

_mesh = plsc.VectorSubcoreMesh(core_axis_name="c", subcore_axis_name="s")


@functools.partial(
    pl.kernel,
    mesh=_mesh,
    compiler_params=pltpu.CompilerParams(needs_layout_passes=False),
    out_type=jax.ShapeDtypeStruct((NTIMES * FEATS,), jnp.float32),
    scratch_types=[
        pltpu.VMEM((CPW,), jnp.float32),
        pltpu.VMEM((ROWS * FEATS,), jnp.float32),
        pltpu.VMEM((L * FEATS,), jnp.float32),
    ],
)
def _sc_interp(times_hbm, table_hbm, out_hbm, times_v, table_v, out_v):
    wid = lax.axis_index("s") * NC + lax.axis_index("c")
    base = wid * CPW
    pltpu.sync_copy(times_hbm.at[pl.ds(base, CPW)], times_v)
    pltpu.sync_copy(table_hbm, table_v)

    orow = lax.iota(jnp.int32, L) * FEATS

    def group_body(g, carry):
        t = times_v[pl.ds(g * L, L)]
        data = t * float(ROWS)
        li = jnp.clip(data, 0.0, float(ROWS - 1)).astype(jnp.int32)
        ri = jnp.minimum(li + 1, ROWS - 1)
        lw = data - li.astype(jnp.float32)
        rw = 1.0 - lw
        lbase = li * FEATS
        rbase = ri * FEATS

        @plsc.parallel_loop(0, FEATS, unroll=8)
        def col_body(c):
            cs = jnp.full((L,), c, dtype=jnp.int32)
            le = plsc.load_gather(table_v, [lbase + cs])
            re = plsc.load_gather(table_v, [rbase + cs])
            o = rw * le + lw * re
            plsc.store_scatter(out_v, [orow + cs], o)
        pltpu.sync_copy(out_v, out_hbm.at[pl.ds((base + g * L) * FEATS, L * FEATS)])
        return carry

    lax.fori_loop(0, NGROUPS, group_body, 0)


def kernel(times, table):
    out = _sc_interp(times, table.reshape(ROWS * FEATS))
    return out.reshape(NTIMES, FEATS)


# speedup vs baseline: 5.9999x; 3.3399x over previous
import functools

import jax
import jax.numpy as jnp
from jax import lax
from jax.experimental import pallas as pl
from jax.experimental.pallas import tpu as pltpu
from jax.experimental.pallas import tpu_sc as plsc

FEATS = 512
ROWS = 16
NTIMES = 16384
NC, NS, L = 2, 16, 16
NW = NC * NS
CPW = NTIMES // NW
NGROUPS = CPW // L
EBLK = 32
NBLK = CPW // EBLK
NCH = FEATS // L

_mesh = plsc.VectorSubcoreMesh(core_axis_name="c", subcore_axis_name="s")


@functools.partial(
    pl.kernel,
    mesh=_mesh,
    compiler_params=pltpu.CompilerParams(needs_layout_passes=False),
    out_type=jax.ShapeDtypeStruct((NTIMES * FEATS,), jnp.float32),
    scratch_types=[
        pltpu.VMEM((CPW,), jnp.float32),
        pltpu.VMEM((ROWS * FEATS,), jnp.float32),
        pltpu.VMEM((CPW + L,), jnp.int32),
        pltpu.VMEM((CPW + L,), jnp.int32),
        pltpu.VMEM((CPW + L,), jnp.float32),
        pltpu.VMEM((CPW + L,), jnp.float32),
        pltpu.VMEM((EBLK * FEATS,), jnp.float32),
    ],
)
def _sc_interp(times_hbm, table_hbm, out_hbm,
               times_v, table_v, lb_v, rb_v, lw_v, rw_v, out_v):
    wid = lax.axis_index("s") * NC + lax.axis_index("c")
    base = wid * CPW
    pltpu.sync_copy(times_hbm.at[pl.ds(base, CPW)], times_v)
    pltpu.sync_copy(table_hbm, table_v)

    @plsc.parallel_loop(0, NGROUPS, unroll=4)
    def weight_body(g):
        t = times_v[pl.ds(g * L, L)]
        data = t * float(ROWS)
        li = jnp.clip(data, 0.0, float(ROWS - 1)).astype(jnp.int32)
        ri = jnp.minimum(li + 1, ROWS - 1)
        lw = data - li.astype(jnp.float32)
        sl = pl.ds(g * L, L)
        lb_v[sl] = li * FEATS
        rb_v[sl] = ri * FEATS
        lw_v[sl] = lw
        rw_v[sl] = 1.0 - lw

    def blk_body(b, carry):
        eb = b * EBLK

        @plsc.parallel_loop(0, EBLK, unroll=2)
        def elem_body(e):
            lb = lb_v[pl.ds(eb + e, L)][0]
            rb = rb_v[pl.ds(eb + e, L)][0]
            lwv = jnp.full((L,), lw_v[pl.ds(eb + e, L)][0], dtype=jnp.float32)
            rwv = jnp.full((L,), rw_v[pl.ds(eb + e, L)][0], dtype=jnp.float32)
            ob = e * FEATS
            for k in range(NCH):
                le = table_v[pl.ds(lb + k * L, L)]
                re = table_v[pl.ds(rb + k * L, L)]
                out_v[pl.ds(ob + k * L, L)] = rwv * le + lwv * re

        pltpu.sync_copy(out_v, out_hbm.at[pl.ds((base + eb) * FEATS, EBLK * FEATS)])
        return carry

    lax.fori_loop(0, NBLK, blk_body, 0)


def kernel(times, table):
    out = _sc_interp(times, table.reshape(ROWS * FEATS))
    return out.reshape(NTIMES, FEATS)
